# baseline (speedup 1.0000x reference)
import functools

import jax
import jax.numpy as jnp
from jax import lax
from jax.experimental import pallas as pl
from jax.experimental.pallas import tpu as pltpu
from jax.experimental.pallas import tpu_sc as plsc

_K = 8
_WPB = 4
_CH = 32


def _sc_combine(x, t, n, ac, om):
    B, C, H, W = x.shape
    K0 = B - _K
    rows_per_batch = C * H
    rows_per_worker = rows_per_batch // _WPB
    n_chunks = rows_per_worker // _CH
    info = plsc.get_sparse_core_info()
    nc = info.num_cores
    mesh = plsc.VectorSubcoreMesh(core_axis_name="c", subcore_axis_name="s")

    @functools.partial(
        pl.kernel,
        mesh=mesh,
        out_type=jax.ShapeDtypeStruct((_K, C, H, W), jnp.float32),
        scratch_types=[
            pltpu.VMEM((16,), jnp.int32),
            pltpu.VMEM((16,), jnp.float32),
            pltpu.VMEM((16,), jnp.float32),
            pltpu.VMEM((_CH, W), jnp.float32),
            pltpu.VMEM((_CH, W), jnp.float32),
            pltpu.VMEM((_CH, W), jnp.float32),
            pltpu.SemaphoreType.DMA,
        ],
    )
    def sc_kernel(x_hbm, t_hbm, n_hbm, ac_hbm, om_hbm, o_hbm,
                  idx_v, c1_v, c2_v, xbuf, nbuf, obuf, sem):
        wid = lax.axis_index("s") * nc + lax.axis_index("c")
        b = K0 + wid // _WPB
        q = wid % _WPB

        bvec = jnp.zeros((16,), jnp.int32) + b.astype(jnp.int32)
        pltpu.async_copy(t_hbm.at[bvec], idx_v, sem).wait()
        pltpu.async_copy(ac_hbm.at[idx_v], c1_v, sem).wait()
        pltpu.async_copy(om_hbm.at[idx_v], c2_v, sem).wait()
        c1 = c1_v[...]
        c2 = c2_v[...]

        def chunk_body(k, carry):
            g = q * rows_per_worker + k * _CH
            c = g // H
            h = g - c * H
            pltpu.sync_copy(x_hbm.at[b, c, pl.ds(h, _CH)], xbuf)
            pltpu.sync_copy(n_hbm.at[b, c, pl.ds(h, _CH)], nbuf)

            def row_body(r, rc):
                for j in range(W // 16):
                    sl = pl.ds(j * 16, 16)
                    obuf[r, sl] = c1 * xbuf[r, sl] + c2 * nbuf[r, sl]
                return rc

            lax.fori_loop(0, _CH, row_body, 0)
            pltpu.sync_copy(obuf, o_hbm.at[b - K0, c, pl.ds(h, _CH)])
            return carry

        lax.fori_loop(0, n_chunks, chunk_body, 0)

    return sc_kernel(x, t, n, ac, om)


def _combine_body(t_ref, ac_ref, om_ref, x_ref, n_ref, o_ref):
    b = pl.program_id(0)
    tt = t_ref[b]
    c1 = ac_ref[tt]
    c2 = om_ref[tt]
    o_ref[...] = c1 * x_ref[...] + c2 * n_ref[...]


def kernel(x_start, t, noise, sqrt_alphas_cumprod, sqrt_one_minus_alphas_cumprod):
    B, C, H, W = x_start.shape
    K0 = B - _K
    t32 = t.astype(jnp.int32)

    out_sc = _sc_combine(x_start, t32, noise, sqrt_alphas_cumprod,
                         sqrt_one_minus_alphas_cumprod)

    smem = pl.BlockSpec(memory_space=pltpu.SMEM)
    blk = pl.BlockSpec((1, C, H, W), lambda b: (b, 0, 0, 0))

    out_tc = pl.pallas_call(
        _combine_body,
        grid=(K0,),
        in_specs=[smem, smem, smem, blk, blk],
        out_specs=blk,
        out_shape=jax.ShapeDtypeStruct((B, C, H, W), jnp.float32),
    )(t32, sqrt_alphas_cumprod, sqrt_one_minus_alphas_cumprod,
      x_start, noise)

    return lax.dynamic_update_slice(out_tc, out_sc, (K0, 0, 0, 0))


# speedup vs baseline: 1.0330x; 1.0330x over previous
import functools

import jax
import jax.numpy as jnp
from jax import lax
from jax.experimental import pallas as pl
from jax.experimental.pallas import tpu as pltpu
from jax.experimental.pallas import tpu_sc as plsc

_K = 8
_WPB = 4
_CH = 32


def _sc_combine(x, t, n, ac, om):
    B, C, H, W = x.shape
    K0 = B - _K
    rows_per_batch = C * H
    rows_per_worker = rows_per_batch // _WPB
    n_chunks = rows_per_worker // _CH
    n_pairs = n_chunks // 2
    info = plsc.get_sparse_core_info()
    nc = info.num_cores
    mesh = plsc.VectorSubcoreMesh(core_axis_name="c", subcore_axis_name="s")

    @functools.partial(
        pl.kernel,
        mesh=mesh,
        out_type=jax.ShapeDtypeStruct((_K, C, H, W), jnp.float32),
        scratch_types=[
            pltpu.VMEM((16,), jnp.int32),
            pltpu.VMEM((16,), jnp.float32),
            pltpu.VMEM((16,), jnp.float32),
            pltpu.VMEM((_CH, W), jnp.float32),
            pltpu.VMEM((_CH, W), jnp.float32),
            pltpu.VMEM((_CH, W), jnp.float32),
            pltpu.VMEM((_CH, W), jnp.float32),
            pltpu.VMEM((_CH, W), jnp.float32),
            pltpu.VMEM((_CH, W), jnp.float32),
            pltpu.SemaphoreType.DMA,
            pltpu.SemaphoreType.DMA,
            pltpu.SemaphoreType.DMA,
            pltpu.SemaphoreType.DMA,
            pltpu.SemaphoreType.DMA,
            pltpu.SemaphoreType.DMA,
            pltpu.SemaphoreType.DMA,
        ],
    )
    def sc_kernel(x_hbm, t_hbm, n_hbm, ac_hbm, om_hbm, o_hbm,
                  idx_v, c1_v, c2_v,
                  xb0, xb1, nb0, nb1, ob0, ob1,
                  sg, sx0, sx1, sn0, sn1, so0, so1):
        wid = lax.axis_index("s") * nc + lax.axis_index("c")
        b = K0 + wid // _WPB
        q = wid % _WPB

        bvec = jnp.zeros((16,), jnp.int32) + b.astype(jnp.int32)
        pltpu.async_copy(t_hbm.at[bvec], idx_v, sg).wait()
        pltpu.async_copy(ac_hbm.at[idx_v], c1_v, sg).wait()
        pltpu.async_copy(om_hbm.at[idx_v], c2_v, sg).wait()
        c1 = c1_v[...]
        c2 = c2_v[...]

        def addr(k):
            g = q * rows_per_worker + k * _CH
            c = g // H
            return c, g - c * H

        def x_src(k):
            c, h = addr(k)
            return x_hbm.at[b, c, pl.ds(h, _CH)]

        def n_src(k):
            c, h = addr(k)
            return n_hbm.at[b, c, pl.ds(h, _CH)]

        def o_dst(k):
            c, h = addr(k)
            return o_hbm.at[b - K0, c, pl.ds(h, _CH)]

        def combine(xb, nb, ob):
            def row_body(r, rc):
                for j in range(W // 16):
                    sl = pl.ds(j * 16, 16)
                    ob[r, sl] = c1 * xb[r, sl] + c2 * nb[r, sl]
                return rc
            lax.fori_loop(0, _CH, row_body, 0)

        pltpu.async_copy(x_src(0), xb0, sx0)
        pltpu.async_copy(n_src(0), nb0, sn0)

        def pair_body(gp, carry):
            k0 = 2 * gp
            k1 = k0 + 1
            k2 = k0 + 2

            pltpu.make_async_copy(x_src(k0), xb0, sx0).wait()
            pltpu.make_async_copy(n_src(k0), nb0, sn0).wait()
            pltpu.async_copy(x_src(k1), xb1, sx1)
            pltpu.async_copy(n_src(k1), nb1, sn1)

            @pl.when(gp > 0)
            def _():
                pltpu.make_async_copy(ob0, o_dst(k0 - 2), so0).wait()

            combine(xb0, nb0, ob0)
            pltpu.async_copy(ob0, o_dst(k0), so0)

            pltpu.make_async_copy(x_src(k1), xb1, sx1).wait()
            pltpu.make_async_copy(n_src(k1), nb1, sn1).wait()

            @pl.when(k2 < n_chunks)
            def _():
                pltpu.async_copy(x_src(k2), xb0, sx0)
                pltpu.async_copy(n_src(k2), nb0, sn0)

            @pl.when(gp > 0)
            def _():
                pltpu.make_async_copy(ob1, o_dst(k1 - 2), so1).wait()

            combine(xb1, nb1, ob1)
            pltpu.async_copy(ob1, o_dst(k1), so1)
            return carry

        lax.fori_loop(0, n_pairs, pair_body, 0)

        pltpu.make_async_copy(ob0, o_dst(n_chunks - 2), so0).wait()
        pltpu.make_async_copy(ob1, o_dst(n_chunks - 1), so1).wait()

    return sc_kernel(x, t, n, ac, om)


def _combine_body(t_ref, ac_ref, om_ref, x_ref, n_ref, o_ref):
    b = pl.program_id(0)
    tt = t_ref[b]
    c1 = ac_ref[tt]
    c2 = om_ref[tt]
    o_ref[...] = c1 * x_ref[...] + c2 * n_ref[...]


def kernel(x_start, t, noise, sqrt_alphas_cumprod, sqrt_one_minus_alphas_cumprod):
    B, C, H, W = x_start.shape
    K0 = B - _K
    t32 = t.astype(jnp.int32)

    out_sc = _sc_combine(x_start, t32, noise, sqrt_alphas_cumprod,
                         sqrt_one_minus_alphas_cumprod)

    smem = pl.BlockSpec(memory_space=pltpu.SMEM)
    blk = pl.BlockSpec((1, C, H, W), lambda b: (b, 0, 0, 0))

    out_tc = pl.pallas_call(
        _combine_body,
        grid=(K0,),
        in_specs=[smem, smem, smem, blk, blk],
        out_specs=blk,
        out_shape=jax.ShapeDtypeStruct((B, C, H, W), jnp.float32),
    )(t32, sqrt_alphas_cumprod, sqrt_one_minus_alphas_cumprod,
      x_start, noise)

    return lax.dynamic_update_slice(out_tc, out_sc, (K0, 0, 0, 0))


# speedup vs baseline: 1.4324x; 1.3867x over previous
import jax
import jax.numpy as jnp
from jax import lax
from jax.experimental import pallas as pl
from jax.experimental.pallas import tpu as pltpu

_D = 3


def _combine_body(t_ref, ac_ref, om_ref, x_hbm, n_hbm, o_hbm,
                  xb, nb, ob, xsem, nsem, osem):
    i = pl.program_id(0)
    nsteps = pl.num_programs(0)
    s = lax.rem(i, _D)

    @pl.when(i == 0)
    def _():
        for k in range(_D):
            pltpu.make_async_copy(x_hbm.at[k], xb.at[k], xsem.at[k]).start()
            pltpu.make_async_copy(n_hbm.at[k], nb.at[k], nsem.at[k]).start()

    pltpu.make_async_copy(x_hbm.at[i], xb.at[s], xsem.at[s]).wait()
    pltpu.make_async_copy(n_hbm.at[i], nb.at[s], nsem.at[s]).wait()

    @pl.when(i >= _D)
    def _():
        pltpu.make_async_copy(ob.at[s], o_hbm.at[i - _D], osem.at[s]).wait()

    tt = t_ref[i]
    c1 = ac_ref[tt]
    c2 = om_ref[tt]
    ob[s] = c1 * xb[s] + c2 * nb[s]

    pltpu.make_async_copy(ob.at[s], o_hbm.at[i], osem.at[s]).start()

    @pl.when(i + _D < nsteps)
    def _():
        pltpu.make_async_copy(x_hbm.at[i + _D], xb.at[s], xsem.at[s]).start()
        pltpu.make_async_copy(n_hbm.at[i + _D], nb.at[s], nsem.at[s]).start()

    @pl.when(i == nsteps - 1)
    def _():
        for k in range(_D):
            j = nsteps - _D + k
            pltpu.make_async_copy(ob.at[j % _D], o_hbm.at[j],
                                  osem.at[j % _D]).wait()


def kernel(x_start, t, noise, sqrt_alphas_cumprod, sqrt_one_minus_alphas_cumprod):
    B, C, H, W = x_start.shape

    smem = pl.BlockSpec(memory_space=pltpu.SMEM)
    hbm = pl.BlockSpec(memory_space=pltpu.MemorySpace.HBM)

    out = pl.pallas_call(
        _combine_body,
        grid=(B,),
        in_specs=[smem, smem, smem, hbm, hbm],
        out_specs=hbm,
        out_shape=jax.ShapeDtypeStruct((B, C, H, W), jnp.float32),
        scratch_shapes=[
            pltpu.VMEM((_D, C, H, W), jnp.float32),
            pltpu.VMEM((_D, C, H, W), jnp.float32),
            pltpu.VMEM((_D, C, H, W), jnp.float32),
            pltpu.SemaphoreType.DMA((_D,)),
            pltpu.SemaphoreType.DMA((_D,)),
            pltpu.SemaphoreType.DMA((_D,)),
        ],
    )(t.astype(jnp.int32), sqrt_alphas_cumprod, sqrt_one_minus_alphas_cumprod,
      x_start, noise)
    return out
